# baseline (speedup 1.0000x reference)
import functools
import math

import jax
import jax.numpy as jnp
from jax import lax
from jax.experimental import pallas as pl
from jax.experimental.pallas import tpu as pltpu
from jax.experimental.pallas import tpu_sc as plsc

_NUM_ROWS = 1024
_RPB = 16
_NUM_BLOCKS = 64
_D = 4096
_SCALE = math.sqrt(_NUM_ROWS / _RPB)
_LANES = 16

_NW = 32
_ROWS_PER_W = _NUM_ROWS // _NW
_CHUNK_ROWS = 4
_NCHUNKS = _ROWS_PER_W // _CHUNK_ROWS
_CPB = _RPB // _CHUNK_ROWS
_NBUF = 6
_UNROLL = 8


def _make_sc_kernel():
    mesh = plsc.VectorSubcoreMesh(core_axis_name="c", subcore_axis_name="s")

    @functools.partial(
        pl.kernel,
        mesh=mesh,
        out_type=jax.ShapeDtypeStruct((_NUM_BLOCKS, _RPB, _D), jnp.float32),
        scratch_types=(
            [pltpu.VMEM((_CHUNK_ROWS, _D), jnp.float32)] * _NBUF
            + [pltpu.SemaphoreType.DMA] * (2 * _NBUF)
        ),
    )
    def sc_kernel(in_hbm, out_hbm, *scratch):
        bufs = scratch[:_NBUF]
        in_sems = scratch[_NBUF:2 * _NBUF]
        out_sems = scratch[2 * _NBUF:]
        wid = lax.axis_index("s") * 2 + lax.axis_index("c")
        row0 = wid * _ROWS_PER_W
        even_scale = jnp.where((wid % 4) == 0, 0.0, _SCALE).astype(jnp.float32)

        def start_in(g):
            src = in_hbm.at[pl.ds(row0 + g * _CHUNK_ROWS, _CHUNK_ROWS), :]
            return pltpu.async_copy(src, bufs[g % _NBUF], in_sems[g % _NBUF])

        def start_out(g):
            block = wid * 2 + (g // _CPB)
            dst = out_hbm.at[block, pl.ds((g % _CPB) * _CHUNK_ROWS, _CHUNK_ROWS), :]
            return pltpu.async_copy(bufs[g % _NBUF], dst, out_sems[g % _NBUF])

        in_handles = {g: start_in(g) for g in range(_NBUF)}
        out_handles = {}
        waited_out = set()
        for g in range(_NCHUNKS):
            nxt = g + _NBUF - 1
            if nxt >= _NBUF and nxt < _NCHUNKS:
                out_handles[nxt - _NBUF].wait()
                waited_out.add(nxt - _NBUF)
                in_handles[nxt] = start_in(nxt)
            in_handles[g].wait()
            buf = bufs[g % _NBUF]
            scale = even_scale if g < _CPB else jnp.float32(_SCALE)

            @plsc.parallel_loop(0, _D, step=_LANES, unroll=_UNROLL)
            def scale_body(i, buf=buf, scale=scale):
                for r in range(_CHUNK_ROWS):
                    sl = pl.ds(i, _LANES)
                    buf[r, sl] = buf[r, sl] * scale

            out_handles[g] = start_out(g)
        for g in range(_NCHUNKS):
            if g not in waited_out:
                out_handles[g].wait()

    return sc_kernel


_sc_kernel = _make_sc_kernel()


@jax.jit
def kernel(inputs):
    return _sc_kernel(inputs)


# speedup vs baseline: 1.0136x; 1.0136x over previous
import functools
import math

import jax
import jax.numpy as jnp
from jax import lax
from jax.experimental import pallas as pl
from jax.experimental.pallas import tpu as pltpu
from jax.experimental.pallas import tpu_sc as plsc

_NUM_ROWS = 1024
_RPB = 16
_NUM_BLOCKS = 64
_D = 4096
_SCALE = math.sqrt(_NUM_ROWS / _RPB)
_LANES = 16

_NW = 32
_ROWS_PER_W = _NUM_ROWS // _NW
_CHUNK_ROWS = 2
_NCHUNKS = _ROWS_PER_W // _CHUNK_ROWS
_CPB = _RPB // _CHUNK_ROWS
_NBUF = 12
_UNROLL = 8


def _make_sc_kernel():
    mesh = plsc.VectorSubcoreMesh(core_axis_name="c", subcore_axis_name="s")

    @functools.partial(
        pl.kernel,
        mesh=mesh,
        out_type=jax.ShapeDtypeStruct((_NUM_BLOCKS, _RPB, _D), jnp.float32),
        scratch_types=(
            [pltpu.VMEM((_CHUNK_ROWS, _D), jnp.float32)] * _NBUF
            + [pltpu.SemaphoreType.DMA] * (2 * _NBUF)
        ),
    )
    def sc_kernel(in_hbm, out_hbm, *scratch):
        bufs = scratch[:_NBUF]
        in_sems = scratch[_NBUF:2 * _NBUF]
        out_sems = scratch[2 * _NBUF:]
        wid = lax.axis_index("s") * 2 + lax.axis_index("c")
        row0 = wid * _ROWS_PER_W
        even_scale = jnp.where((wid % 4) == 0, 0.0, _SCALE).astype(jnp.float32)

        def start_in(g):
            src = in_hbm.at[pl.ds(row0 + g * _CHUNK_ROWS, _CHUNK_ROWS), :]
            return pltpu.async_copy(src, bufs[g % _NBUF], in_sems[g % _NBUF])

        def start_out(g):
            block = wid * 2 + (g // _CPB)
            dst = out_hbm.at[block, pl.ds((g % _CPB) * _CHUNK_ROWS, _CHUNK_ROWS), :]
            return pltpu.async_copy(bufs[g % _NBUF], dst, out_sems[g % _NBUF])

        in_handles = {g: start_in(g) for g in range(_NBUF)}
        out_handles = {}
        waited_out = set()
        for g in range(_NCHUNKS):
            nxt = g + _NBUF - 1
            if nxt >= _NBUF and nxt < _NCHUNKS:
                out_handles[nxt - _NBUF].wait()
                waited_out.add(nxt - _NBUF)
                in_handles[nxt] = start_in(nxt)
            in_handles[g].wait()
            buf = bufs[g % _NBUF]
            scale = even_scale if g < _CPB else jnp.float32(_SCALE)

            @plsc.parallel_loop(0, _D, step=_LANES, unroll=_UNROLL)
            def scale_body(i, buf=buf, scale=scale):
                for r in range(_CHUNK_ROWS):
                    sl = pl.ds(i, _LANES)
                    buf[r, sl] = buf[r, sl] * scale

            out_handles[g] = start_out(g)
        for g in range(_NCHUNKS):
            if g not in waited_out:
                out_handles[g].wait()

    return sc_kernel


_sc_kernel = _make_sc_kernel()


@jax.jit
def kernel(inputs):
    return _sc_kernel(inputs)
